# baseline (speedup 1.0000x reference)
import jax
import jax.numpy as jnp
from jax import lax
from jax.experimental import pallas as pl
from jax.experimental.pallas import tpu as pltpu
from jax.experimental.pallas import tpu_sc as plsc

N = 10000
E = 320000
H = 128

NC = 2
NS = 16
NW = NC * NS
EW = E // NW
EW2 = 10240
E2 = EW2 * NW
C = 64
CG = 80
KG = EW2 // CG
K2 = EW2 // C
G2 = EW2 // 16
SB = 16
DB = 1280
NP2 = 10240
NSL = NP2 // NS

BE = 1280
GL = BE // 128
NB = E2 // BE


def _silu(x):
    return x / (1.0 + jnp.exp(-x))


def _precompute_body(h_ref, wa_ref, wb_ref, xa_ref, xb_ref):
    h = h_ref[...]
    xa_ref[...] = jnp.dot(h, wa_ref[...], preferred_element_type=jnp.float32)
    xb_ref[...] = jnp.dot(h, wb_ref[...], preferred_element_type=jnp.float32)


def _precompute(h, WaT, WbT):
    blk = 2000
    return pl.pallas_call(
        _precompute_body,
        grid=(N // blk,),
        in_specs=[
            pl.BlockSpec((blk, H), lambda i: (i, 0)),
            pl.BlockSpec((H, H), lambda i: (0, 0)),
            pl.BlockSpec((H, H), lambda i: (0, 0)),
        ],
        out_specs=[
            pl.BlockSpec((blk, H), lambda i: (i, 0)),
            pl.BlockSpec((blk, H), lambda i: (i, 0)),
        ],
        out_shape=[
            jax.ShapeDtypeStruct((N, H), jnp.float32),
            jax.ShapeDtypeStruct((N, H), jnp.float32),
        ],
    )(h, WaT, WbT)


def _gather_body(xa_hbm, xb_hbm, px_hbm, py_hbm, pz_hbm, rowg_hbm, colg_hbm,
                 u_hbm, dfx_hbm, dfy_hbm, dfz_hbm,
                 ridx, cidx, posx, posy, posz,
                 bufa0, bufb0, bufa1, bufb1, dbx, dby, dbz,
                 gsem0, gsem1, wsem0, wsem1):
    wid = lax.axis_index("s") * NC + lax.axis_index("c")
    base = wid * EW2
    pltpu.sync_copy(rowg_hbm.at[pl.ds(base, EW2)], ridx)
    pltpu.sync_copy(colg_hbm.at[pl.ds(base, EW2)], cidx)
    pltpu.sync_copy(px_hbm, posx)
    pltpu.sync_copy(py_hbm, posy)
    pltpu.sync_copy(pz_hbm, posz)

    def diff_group(g, carry):
        o = g * 16
        ir = ridx[pl.ds(o, 16)]
        ic = cidx[pl.ds(o, 16)]
        dbx[pl.ds(o, 16)] = (plsc.load_gather(posx, [ir])
                             - plsc.load_gather(posx, [ic]))
        dby[pl.ds(o, 16)] = (plsc.load_gather(posy, [ir])
                             - plsc.load_gather(posy, [ic]))
        dbz[pl.ds(o, 16)] = (plsc.load_gather(posz, [ir])
                             - plsc.load_gather(posz, [ic]))
        return carry

    lax.fori_loop(0, G2, diff_group, 0)
    pltpu.sync_copy(dbx, dfx_hbm.at[pl.ds(base, EW2)])
    pltpu.sync_copy(dby, dfy_hbm.at[pl.ds(base, EW2)])
    pltpu.sync_copy(dbz, dfz_hbm.at[pl.ds(base, EW2)])

    bufs = ((bufa0, bufb0, gsem0, wsem0), (bufa1, bufb1, gsem1, wsem1))

    def fire_gather(k, ba, bb, gs):
        ko = k * CG
        pltpu.make_async_copy(xa_hbm.at[ridx.at[pl.ds(ko, CG)]], ba,
                              gs).start()
        pltpu.make_async_copy(xb_hbm.at[cidx.at[pl.ds(ko, CG)]], bb,
                              gs).start()

    def add_rows(ba, bb):
        def body(i, carry):
            r = i // 8
            o = (i % 8) * 16
            ba[r, pl.ds(o, 16)] = ba[r, pl.ds(o, 16)] + bb[r, pl.ds(o, 16)]
            return carry
        lax.fori_loop(0, CG * 8, body, 0)

    fire_gather(0, bufa0, bufb0, gsem0)

    def step(j, carry):
        for b in (0, 1):
            k = 2 * j + b
            ba, bb, gs, ws = bufs[b]
            nba, nbb, ngs, nws = bufs[1 - b]

            @pl.when(k >= 1)
            def _():
                pltpu.make_async_copy(
                    nba, u_hbm.at[pl.ds(base, CG)], nws).wait()

            @pl.when(k < KG - 1)
            def _():
                fire_gather(k + 1, nba, nbb, ngs)

            pltpu.make_async_copy(
                xa_hbm.at[ridx.at[pl.ds(k * CG, CG)]], ba, gs).wait()
            pltpu.make_async_copy(
                xb_hbm.at[cidx.at[pl.ds(k * CG, CG)]], bb, gs).wait()
            add_rows(ba, bb)
            pltpu.make_async_copy(
                ba, u_hbm.at[pl.ds(base + k * CG, CG)], ws).start()
        return carry

    lax.fori_loop(0, KG // 2, step, 0)
    pltpu.make_async_copy(bufa1, u_hbm.at[pl.ds(base, CG)], wsem1).wait()


def _gather(Xa, Xb, px, py, pz, rowg, colg):
    mesh = plsc.VectorSubcoreMesh(core_axis_name="c", subcore_axis_name="s")
    f = pl.kernel(
        _gather_body,
        mesh=mesh,
        out_type=[
            jax.ShapeDtypeStruct((E2, H), jnp.float32),
            jax.ShapeDtypeStruct((E2,), jnp.float32),
            jax.ShapeDtypeStruct((E2,), jnp.float32),
            jax.ShapeDtypeStruct((E2,), jnp.float32),
        ],
        scratch_types=[
            pltpu.VMEM((EW2,), jnp.int32),
            pltpu.VMEM((EW2,), jnp.int32),
            pltpu.VMEM((N,), jnp.float32),
            pltpu.VMEM((N,), jnp.float32),
            pltpu.VMEM((N,), jnp.float32),
            pltpu.VMEM((CG, H), jnp.float32),
            pltpu.VMEM((CG, H), jnp.float32),
            pltpu.VMEM((CG, H), jnp.float32),
            pltpu.VMEM((CG, H), jnp.float32),
            pltpu.VMEM((EW2,), jnp.float32),
            pltpu.VMEM((EW2,), jnp.float32),
            pltpu.VMEM((EW2,), jnp.float32),
            pltpu.SemaphoreType.DMA,
            pltpu.SemaphoreType.DMA,
            pltpu.SemaphoreType.DMA,
            pltpu.SemaphoreType.DMA,
        ],
        compiler_params=pltpu.CompilerParams(needs_layout_passes=False),
    )
    return f(Xa, Xb, px, py, pz, rowg, colg)


def _edge_body(u0_ref, dfx_ref, dfy_ref, dfz_ref,
               w1d_ref, be1_ref, w2t_ref, be2_ref,
               wc1t_ref, bc1_ref, wc2_ref,
               msg_ref, wod_ref):
    f32 = jnp.float32
    dx = dfx_ref[...].reshape(GL, 128)
    dy = dfy_ref[...].reshape(GL, 128)
    dz = dfz_ref[...].reshape(GL, 128)
    d2 = dx * dx + dy * dy + dz * dz
    dist_g = jnp.maximum(jnp.sqrt(d2), 1e-6)
    ones11 = jnp.ones((1, 1), dtype=f32)
    dn = (((0,), (0,)), ((), ()))
    dist_c = jnp.concatenate(
        [lax.dot_general(dist_g[r:r + 1, :], ones11, dn,
                         precision=lax.Precision.HIGHEST,
                         preferred_element_type=f32) for r in range(GL)],
        axis=0)
    u = u0_ref[...] + dist_c * w1d_ref[...] + be1_ref[...]
    e1 = _silu(u)
    msg = _silu(jnp.dot(e1, w2t_ref[...], preferred_element_type=jnp.float32)
                + be2_ref[...])
    msg_ref[...] = msg
    t = _silu(jnp.dot(msg, wc1t_ref[...], preferred_element_type=jnp.float32)
              + bc1_ref[...])
    w = jnp.sum(t * wc2_ref[...], axis=1, keepdims=True)
    w = jnp.clip(w, -1.0, 1.0)
    wod_c = w / dist_c
    ri = lax.broadcasted_iota(jnp.int32, (128, 128), 0)
    ci = lax.broadcasted_iota(jnp.int32, (128, 128), 1)
    eye = jnp.where(ri == ci, 1.0, 0.0).astype(f32)
    wod_g = jnp.concatenate(
        [lax.dot_general(wod_c[r * 128:(r + 1) * 128, :], eye, dn,
                         precision=lax.Precision.HIGHEST,
                         preferred_element_type=f32) for r in range(GL)],
        axis=0)
    wod_ref[...] = wod_g.reshape(1, GL, 128)


def _edge_mlp(U0, dfx, dfy, dfz, w1d, be1, We2T, be2, Wc1T, bc1, Wc2):
    return pl.pallas_call(
        _edge_body,
        grid=(NB,),
        in_specs=[
            pl.BlockSpec((BE, H), lambda i: (i, 0)),
            pl.BlockSpec((1, GL, 128), lambda i: (i, 0, 0)),
            pl.BlockSpec((1, GL, 128), lambda i: (i, 0, 0)),
            pl.BlockSpec((1, GL, 128), lambda i: (i, 0, 0)),
            pl.BlockSpec((1, H), lambda i: (0, 0)),
            pl.BlockSpec((1, H), lambda i: (0, 0)),
            pl.BlockSpec((H, H), lambda i: (0, 0)),
            pl.BlockSpec((1, H), lambda i: (0, 0)),
            pl.BlockSpec((H, H), lambda i: (0, 0)),
            pl.BlockSpec((1, H), lambda i: (0, 0)),
            pl.BlockSpec((1, H), lambda i: (0, 0)),
        ],
        out_specs=[
            pl.BlockSpec((BE, H), lambda i: (i, 0)),
            pl.BlockSpec((1, GL, 128), lambda i: (i, 0, 0)),
        ],
        out_shape=[
            jax.ShapeDtypeStruct((E2, H), jnp.float32),
            jax.ShapeDtypeStruct((NB, GL, 128), jnp.float32),
        ],
    )(U0, dfx.reshape(NB, GL, 128), dfy.reshape(NB, GL, 128),
      dfz.reshape(NB, GL, 128), w1d, be1, We2T, be2, Wc1T, bc1, Wc2)


def _scatter_body(msg_hbm, wod_hbm, dfx_hbm, dfy_hbm, dfz_hbm,
                  row3d_hbm, rows_hbm, zh_hbm, zp_hbm,
                  agg_hbm, px_hbm, py_hbm, pz_hbm,
                  ridx2, ridx, mbuf, dbx, dby, dbz, wodb,
                  paccx, paccy, paccz, agg_sh, lsem):
    c = lax.axis_index("c")
    s = lax.axis_index("s")
    wid = s * NC + c
    base = wid * EW2
    pltpu.sync_copy(zh_hbm.at[pl.ds(s * NSL, NSL)],
                    agg_sh.at[pl.ds(s * NSL, NSL)])
    pltpu.sync_copy(zp_hbm, paccx)
    pltpu.sync_copy(zp_hbm, paccy)
    pltpu.sync_copy(zp_hbm, paccz)
    plsc.subcore_barrier()

    def superchunk(sb, carry):
        pltpu.sync_copy(row3d_hbm.at[wid, pl.ds(sb * SB, SB)], ridx2)

        def chunk(k, carry2):
            off = base + (sb * SB + k) * C
            lm = pltpu.make_async_copy(msg_hbm.at[pl.ds(off, C)], mbuf,
                                       lsem)
            lm.start()
            lm.wait()
            pltpu.sync_copy(mbuf, agg_sh.at[ridx2.at[k]], add=True)
            return carry2

        lax.fori_loop(0, SB, chunk, 0)
        return carry

    lax.fori_loop(0, K2 // SB, superchunk, 0)

    def superdelta(j, carry):
        off = base + j * DB
        pltpu.sync_copy(rows_hbm.at[pl.ds(off, DB)], ridx)
        pltpu.sync_copy(wod_hbm.at[pl.ds(off, DB)], wodb)
        pltpu.sync_copy(dfx_hbm.at[pl.ds(off, DB)], dbx)
        pltpu.sync_copy(dfy_hbm.at[pl.ds(off, DB)], dby)
        pltpu.sync_copy(dfz_hbm.at[pl.ds(off, DB)], dbz)

        def delta_group(g, carry2):
            o = g * 16
            ir = ridx[pl.ds(o, 16)]
            wod16 = wodb[pl.ds(o, 16)]
            plsc.addupdate_scatter(paccx, [ir], dbx[pl.ds(o, 16)] * wod16)
            plsc.addupdate_scatter(paccy, [ir], dby[pl.ds(o, 16)] * wod16)
            plsc.addupdate_scatter(paccz, [ir], dbz[pl.ds(o, 16)] * wod16)
            return carry2

        lax.fori_loop(0, DB // 16, delta_group, 0)
        return carry

    lax.fori_loop(0, EW2 // DB, superdelta, 0)
    plsc.subcore_barrier()
    pltpu.sync_copy(agg_sh.at[pl.ds(s * NSL, NSL)],
                    agg_hbm.at[c, pl.ds(s * NSL, NSL)])
    pltpu.sync_copy(paccx, px_hbm.at[wid, 0])
    pltpu.sync_copy(paccy, py_hbm.at[wid, 0])
    pltpu.sync_copy(paccz, pz_hbm.at[wid, 0])


def _scatter(msg, wod, dfx, dfy, dfz, row3d, rows, zero_h, zero_p):
    mesh = plsc.VectorSubcoreMesh(core_axis_name="c", subcore_axis_name="s")
    f = pl.kernel(
        _scatter_body,
        mesh=mesh,
        out_type=[
            jax.ShapeDtypeStruct((NC, NP2, H), jnp.float32),
            jax.ShapeDtypeStruct((NW, 1, NP2), jnp.float32),
            jax.ShapeDtypeStruct((NW, 1, NP2), jnp.float32),
            jax.ShapeDtypeStruct((NW, 1, NP2), jnp.float32),
        ],
        scratch_types=[
            pltpu.VMEM((SB, C), jnp.int32),
            pltpu.VMEM((DB,), jnp.int32),
            pltpu.VMEM((C, H), jnp.float32),
            pltpu.VMEM((DB,), jnp.float32),
            pltpu.VMEM((DB,), jnp.float32),
            pltpu.VMEM((DB,), jnp.float32),
            pltpu.VMEM((DB,), jnp.float32),
            pltpu.VMEM((NP2,), jnp.float32),
            pltpu.VMEM((NP2,), jnp.float32),
            pltpu.VMEM((NP2,), jnp.float32),
            pltpu.VMEM_SHARED((NP2, H), jnp.float32),
            pltpu.SemaphoreType.DMA,
        ],
        compiler_params=pltpu.CompilerParams(needs_layout_passes=False),
    )
    return f(msg, wod, dfx, dfy, dfz, row3d, rows, zero_h, zero_p)


def _node_body(h_ref, a0_ref, a1_ref, px_ref, py_ref, pz_ref,
               pix_ref, piy_ref, piz_ref,
               wn1at_ref, wn1bt_ref, bn1_ref, wn2t_ref, bn2_ref,
               hout_ref, sx_ref, sy_ref, sz_ref):
    h = h_ref[...]
    agg = a0_ref[...] + a1_ref[...]
    x1 = (jnp.dot(h, wn1at_ref[...], preferred_element_type=jnp.float32)
          + jnp.dot(agg, wn1bt_ref[...], preferred_element_type=jnp.float32)
          + bn1_ref[...])
    hout_ref[...] = h + (jnp.dot(_silu(x1), wn2t_ref[...],
                                 preferred_element_type=jnp.float32)
                         + bn2_ref[...])
    sx_ref[...] = pix_ref[...] + jnp.sum(px_ref[...].reshape(NW, NP2),
                                         axis=0, keepdims=True)
    sy_ref[...] = piy_ref[...] + jnp.sum(py_ref[...].reshape(NW, NP2),
                                         axis=0, keepdims=True)
    sz_ref[...] = piz_ref[...] + jnp.sum(pz_ref[...].reshape(NW, NP2),
                                         axis=0, keepdims=True)


def _node_mlp(h, a0, a1, pacx, pacy, pacz, pix, piy, piz,
              Wn1aT, Wn1bT, bn1, Wn2T, bn2):
    blk = 2000
    return pl.pallas_call(
        _node_body,
        grid=(N // blk,),
        in_specs=[
            pl.BlockSpec((blk, H), lambda i: (i, 0)),
            pl.BlockSpec((blk, H), lambda i: (i, 0)),
            pl.BlockSpec((blk, H), lambda i: (i, 0)),
            pl.BlockSpec((NW, 1, NP2), lambda i: (0, 0, 0)),
            pl.BlockSpec((NW, 1, NP2), lambda i: (0, 0, 0)),
            pl.BlockSpec((NW, 1, NP2), lambda i: (0, 0, 0)),
            pl.BlockSpec((1, NP2), lambda i: (0, 0)),
            pl.BlockSpec((1, NP2), lambda i: (0, 0)),
            pl.BlockSpec((1, NP2), lambda i: (0, 0)),
            pl.BlockSpec((H, H), lambda i: (0, 0)),
            pl.BlockSpec((H, H), lambda i: (0, 0)),
            pl.BlockSpec((1, H), lambda i: (0, 0)),
            pl.BlockSpec((H, H), lambda i: (0, 0)),
            pl.BlockSpec((1, H), lambda i: (0, 0)),
        ],
        out_specs=[
            pl.BlockSpec((blk, H), lambda i: (i, 0)),
            pl.BlockSpec((1, NP2), lambda i: (0, 0)),
            pl.BlockSpec((1, NP2), lambda i: (0, 0)),
            pl.BlockSpec((1, NP2), lambda i: (0, 0)),
        ],
        out_shape=[
            jax.ShapeDtypeStruct((N, H), jnp.float32),
            jax.ShapeDtypeStruct((1, NP2), jnp.float32),
            jax.ShapeDtypeStruct((1, NP2), jnp.float32),
            jax.ShapeDtypeStruct((1, NP2), jnp.float32),
        ],
    )(h, a0, a1, pacx, pacy, pacz, pix, piy, piz,
      Wn1aT, Wn1bT, bn1, Wn2T, bn2)


def kernel(h, pos, edge_index, We1, be1, We2, be2, Wn1, bn1, Wn2, bn2,
           Wc1, bc1, Wc2):
    f32 = jnp.float32
    i32 = jnp.int32
    WaT = We1[:, :H].T
    WbT = We1[:, H:2 * H].T
    w1d = We1[:, 2 * H].reshape(1, H)
    We2T = We2.T
    Wc1T = Wc1.T
    Wn1aT = Wn1[:, :H].T
    Wn1bT = Wn1[:, H:].T
    Wn2T = Wn2.T
    be1r = be1.reshape(1, H)
    be2r = be2.reshape(1, H)
    bc1r = bc1.reshape(1, H)
    bn1r = bn1.reshape(1, H)
    bn2r = bn2.reshape(1, H)
    Wc2r = Wc2.reshape(1, H)

    px = pos[:, 0]
    py = pos[:, 1]
    pz = pos[:, 2]
    pad = EW2 - EW
    rw = edge_index[0].reshape(NW, EW)
    cw = edge_index[1].reshape(NW, EW)
    zpad = jnp.zeros((NW, pad), dtype=i32)
    rowg = jnp.concatenate([rw, zpad], axis=1).reshape(E2)
    colg = jnp.concatenate([cw, zpad], axis=1).reshape(E2)
    rows = jnp.concatenate([rw, jnp.full((NW, pad), N, dtype=i32)],
                           axis=1).reshape(E2)
    row3d = rows.reshape(NW, K2, C)
    zero_h = jnp.zeros((NP2, H), dtype=f32)
    zero_p = jnp.zeros((NP2,), dtype=f32)
    pixp = jnp.pad(px, (0, NP2 - N)).reshape(1, NP2)
    piyp = jnp.pad(py, (0, NP2 - N)).reshape(1, NP2)
    pizp = jnp.pad(pz, (0, NP2 - N)).reshape(1, NP2)

    Xa, Xb = _precompute(h, WaT, WbT)
    U0, dfx, dfy, dfz = _gather(Xa, Xb, px, py, pz, rowg, colg)
    msg, wod_g = _edge_mlp(U0, dfx, dfy, dfz, w1d, be1r, We2T, be2r,
                           Wc1T, bc1r, Wc2r)
    wod = wod_g.reshape(E2)
    agg2, pacx, pacy, pacz = _scatter(msg, wod, dfx, dfy, dfz, row3d, rows,
                                      zero_h, zero_p)
    h_out, sx, sy, sz = _node_mlp(h, agg2[0, :N], agg2[1, :N],
                                  pacx, pacy, pacz, pixp, piyp, pizp,
                                  Wn1aT, Wn1bT, bn1r, Wn2T, bn2r)
    pos_out = jnp.concatenate([sx[0, :N].reshape(N, 1),
                               sy[0, :N].reshape(N, 1),
                               sz[0, :N].reshape(N, 1)], axis=1)
    return (h_out, pos_out)


# speedup vs baseline: 1.1540x; 1.1540x over previous
import jax
import jax.numpy as jnp
from jax import lax
from jax.experimental import pallas as pl
from jax.experimental.pallas import tpu as pltpu
from jax.experimental.pallas import tpu_sc as plsc

N = 10000
E = 320000
H = 128

NC = 2
NS = 16
NW = NC * NS
EW = E // NW
EW2 = 10240
E2 = EW2 * NW
C = 64
CG = 80
KG = EW2 // CG
K2 = EW2 // C
G2 = EW2 // 16
SB = 16
DB = 1280
NP2 = 10240
NSL = NP2 // NS

BE = 1280
GL = BE // 128
NB = E2 // BE


def _silu(x):
    return x / (1.0 + jnp.exp(-x))


def _precompute_body(h_ref, wa_ref, wb_ref, xa_ref, xb_ref):
    h = h_ref[...]
    xa_ref[...] = jnp.dot(h, wa_ref[...], preferred_element_type=jnp.float32)
    xb_ref[...] = jnp.dot(h, wb_ref[...], preferred_element_type=jnp.float32)


def _precompute(h, WaT, WbT):
    blk = 2000
    return pl.pallas_call(
        _precompute_body,
        grid=(N // blk,),
        in_specs=[
            pl.BlockSpec((blk, H), lambda i: (i, 0)),
            pl.BlockSpec((H, H), lambda i: (0, 0)),
            pl.BlockSpec((H, H), lambda i: (0, 0)),
        ],
        out_specs=[
            pl.BlockSpec((blk, H), lambda i: (i, 0)),
            pl.BlockSpec((blk, H), lambda i: (i, 0)),
        ],
        out_shape=[
            jax.ShapeDtypeStruct((N, H), jnp.float32),
            jax.ShapeDtypeStruct((N, H), jnp.float32),
        ],
    )(h, WaT, WbT)


def _gather_body(xa_hbm, xb_hbm, px_hbm, py_hbm, pz_hbm, rowg_hbm, colg_hbm,
                 u_hbm, dfx_hbm, dfy_hbm, dfz_hbm,
                 ridx, cidx, posx, posy, posz,
                 bufa0, bufb0, bufa1, bufb1, dbx, dby, dbz,
                 gsem0, gsem1, wsem0, wsem1):
    wid = lax.axis_index("s") * NC + lax.axis_index("c")
    base = wid * EW2
    pltpu.sync_copy(rowg_hbm.at[pl.ds(base, EW2)], ridx)
    pltpu.sync_copy(colg_hbm.at[pl.ds(base, EW2)], cidx)
    pltpu.sync_copy(px_hbm, posx)
    pltpu.sync_copy(py_hbm, posy)
    pltpu.sync_copy(pz_hbm, posz)

    def diff_group(g, carry):
        o = g * 16
        ir = ridx[pl.ds(o, 16)]
        ic = cidx[pl.ds(o, 16)]
        dbx[pl.ds(o, 16)] = (plsc.load_gather(posx, [ir])
                             - plsc.load_gather(posx, [ic]))
        dby[pl.ds(o, 16)] = (plsc.load_gather(posy, [ir])
                             - plsc.load_gather(posy, [ic]))
        dbz[pl.ds(o, 16)] = (plsc.load_gather(posz, [ir])
                             - plsc.load_gather(posz, [ic]))
        return carry

    lax.fori_loop(0, G2, diff_group, 0)
    pltpu.sync_copy(dbx, dfx_hbm.at[pl.ds(base, EW2)])
    pltpu.sync_copy(dby, dfy_hbm.at[pl.ds(base, EW2)])
    pltpu.sync_copy(dbz, dfz_hbm.at[pl.ds(base, EW2)])

    bufs = ((bufa0, bufb0, gsem0, wsem0), (bufa1, bufb1, gsem1, wsem1))

    def fire_gather(k, ba, bb, gs):
        ko = k * CG
        pltpu.make_async_copy(xa_hbm.at[ridx.at[pl.ds(ko, CG)]], ba,
                              gs).start()
        pltpu.make_async_copy(xb_hbm.at[cidx.at[pl.ds(ko, CG)]], bb,
                              gs).start()

    def add_rows(ba, bb):
        def body(r, carry):
            for q in range(8):
                o = q * 16
                ba[r, pl.ds(o, 16)] = (ba[r, pl.ds(o, 16)]
                                       + bb[r, pl.ds(o, 16)])
            return carry
        lax.fori_loop(0, CG, body, 0)

    fire_gather(0, bufa0, bufb0, gsem0)

    def step(j, carry):
        for b in (0, 1):
            k = 2 * j + b
            ba, bb, gs, ws = bufs[b]
            nba, nbb, ngs, nws = bufs[1 - b]

            @pl.when(k >= 1)
            def _():
                pltpu.make_async_copy(
                    nba, u_hbm.at[pl.ds(base, CG)], nws).wait()

            @pl.when(k < KG - 1)
            def _():
                fire_gather(k + 1, nba, nbb, ngs)

            pltpu.make_async_copy(
                xa_hbm.at[ridx.at[pl.ds(k * CG, CG)]], ba, gs).wait()
            pltpu.make_async_copy(
                xb_hbm.at[cidx.at[pl.ds(k * CG, CG)]], bb, gs).wait()
            add_rows(ba, bb)
            pltpu.make_async_copy(
                ba, u_hbm.at[pl.ds(base + k * CG, CG)], ws).start()
        return carry

    lax.fori_loop(0, KG // 2, step, 0)
    pltpu.make_async_copy(bufa1, u_hbm.at[pl.ds(base, CG)], wsem1).wait()


def _gather(Xa, Xb, px, py, pz, rowg, colg):
    mesh = plsc.VectorSubcoreMesh(core_axis_name="c", subcore_axis_name="s")
    f = pl.kernel(
        _gather_body,
        mesh=mesh,
        out_type=[
            jax.ShapeDtypeStruct((E2, H), jnp.float32),
            jax.ShapeDtypeStruct((E2,), jnp.float32),
            jax.ShapeDtypeStruct((E2,), jnp.float32),
            jax.ShapeDtypeStruct((E2,), jnp.float32),
        ],
        scratch_types=[
            pltpu.VMEM((EW2,), jnp.int32),
            pltpu.VMEM((EW2,), jnp.int32),
            pltpu.VMEM((N,), jnp.float32),
            pltpu.VMEM((N,), jnp.float32),
            pltpu.VMEM((N,), jnp.float32),
            pltpu.VMEM((CG, H), jnp.float32),
            pltpu.VMEM((CG, H), jnp.float32),
            pltpu.VMEM((CG, H), jnp.float32),
            pltpu.VMEM((CG, H), jnp.float32),
            pltpu.VMEM((EW2,), jnp.float32),
            pltpu.VMEM((EW2,), jnp.float32),
            pltpu.VMEM((EW2,), jnp.float32),
            pltpu.SemaphoreType.DMA,
            pltpu.SemaphoreType.DMA,
            pltpu.SemaphoreType.DMA,
            pltpu.SemaphoreType.DMA,
        ],
        compiler_params=pltpu.CompilerParams(needs_layout_passes=False),
    )
    return f(Xa, Xb, px, py, pz, rowg, colg)


def _edge_body(u0_ref, dfx_ref, dfy_ref, dfz_ref,
               w1d_ref, be1_ref, w2t_ref, be2_ref,
               wc1t_ref, bc1_ref, wc2_ref,
               msg_ref, wod_ref):
    f32 = jnp.float32
    dx = dfx_ref[...].reshape(GL, 128)
    dy = dfy_ref[...].reshape(GL, 128)
    dz = dfz_ref[...].reshape(GL, 128)
    d2 = dx * dx + dy * dy + dz * dz
    dist_g = jnp.maximum(jnp.sqrt(d2), 1e-6)
    ones11 = jnp.ones((1, 1), dtype=f32)
    dn = (((0,), (0,)), ((), ()))
    dist_c = jnp.concatenate(
        [lax.dot_general(dist_g[r:r + 1, :], ones11, dn,
                         precision=lax.Precision.HIGHEST,
                         preferred_element_type=f32) for r in range(GL)],
        axis=0)
    u = u0_ref[...] + dist_c * w1d_ref[...] + be1_ref[...]
    e1 = _silu(u)
    msg = _silu(jnp.dot(e1, w2t_ref[...], preferred_element_type=jnp.float32)
                + be2_ref[...])
    msg_ref[...] = msg
    t = _silu(jnp.dot(msg, wc1t_ref[...], preferred_element_type=jnp.float32)
              + bc1_ref[...])
    w = jnp.sum(t * wc2_ref[...], axis=1, keepdims=True)
    w = jnp.clip(w, -1.0, 1.0)
    wod_c = w / dist_c
    ri = lax.broadcasted_iota(jnp.int32, (128, 128), 0)
    ci = lax.broadcasted_iota(jnp.int32, (128, 128), 1)
    eye = jnp.where(ri == ci, 1.0, 0.0).astype(f32)
    wod_g = jnp.concatenate(
        [lax.dot_general(wod_c[r * 128:(r + 1) * 128, :], eye, dn,
                         precision=lax.Precision.HIGHEST,
                         preferred_element_type=f32) for r in range(GL)],
        axis=0)
    wod_ref[...] = wod_g.reshape(1, GL, 128)


def _edge_mlp(U0, dfx, dfy, dfz, w1d, be1, We2T, be2, Wc1T, bc1, Wc2):
    return pl.pallas_call(
        _edge_body,
        grid=(NB,),
        in_specs=[
            pl.BlockSpec((BE, H), lambda i: (i, 0)),
            pl.BlockSpec((1, GL, 128), lambda i: (i, 0, 0)),
            pl.BlockSpec((1, GL, 128), lambda i: (i, 0, 0)),
            pl.BlockSpec((1, GL, 128), lambda i: (i, 0, 0)),
            pl.BlockSpec((1, H), lambda i: (0, 0)),
            pl.BlockSpec((1, H), lambda i: (0, 0)),
            pl.BlockSpec((H, H), lambda i: (0, 0)),
            pl.BlockSpec((1, H), lambda i: (0, 0)),
            pl.BlockSpec((H, H), lambda i: (0, 0)),
            pl.BlockSpec((1, H), lambda i: (0, 0)),
            pl.BlockSpec((1, H), lambda i: (0, 0)),
        ],
        out_specs=[
            pl.BlockSpec((BE, H), lambda i: (i, 0)),
            pl.BlockSpec((1, GL, 128), lambda i: (i, 0, 0)),
        ],
        out_shape=[
            jax.ShapeDtypeStruct((E2, H), jnp.float32),
            jax.ShapeDtypeStruct((NB, GL, 128), jnp.float32),
        ],
    )(U0, dfx.reshape(NB, GL, 128), dfy.reshape(NB, GL, 128),
      dfz.reshape(NB, GL, 128), w1d, be1, We2T, be2, Wc1T, bc1, Wc2)


def _scatter_body(msg_hbm, wod_hbm, dfx_hbm, dfy_hbm, dfz_hbm,
                  row3d_hbm, rows_hbm, zh_hbm, zp_hbm,
                  agg_hbm, px_hbm, py_hbm, pz_hbm,
                  ridx2, ridx, mbuf, dbx, dby, dbz, wodb,
                  paccx, paccy, paccz, agg_sh, lsem):
    c = lax.axis_index("c")
    s = lax.axis_index("s")
    wid = s * NC + c
    base = wid * EW2
    pltpu.sync_copy(zh_hbm.at[pl.ds(s * NSL, NSL)],
                    agg_sh.at[pl.ds(s * NSL, NSL)])
    pltpu.sync_copy(zp_hbm, paccx)
    pltpu.sync_copy(zp_hbm, paccy)
    pltpu.sync_copy(zp_hbm, paccz)
    plsc.subcore_barrier()

    def superchunk(sb, carry):
        pltpu.sync_copy(row3d_hbm.at[wid, pl.ds(sb * SB, SB)], ridx2)

        def chunk(k, carry2):
            off = base + (sb * SB + k) * C
            lm = pltpu.make_async_copy(msg_hbm.at[pl.ds(off, C)], mbuf,
                                       lsem)
            lm.start()
            lm.wait()
            pltpu.sync_copy(mbuf, agg_sh.at[ridx2.at[k]], add=True)
            return carry2

        lax.fori_loop(0, SB, chunk, 0)
        return carry

    lax.fori_loop(0, K2 // SB, superchunk, 0)

    def superdelta(j, carry):
        off = base + j * DB
        pltpu.sync_copy(rows_hbm.at[pl.ds(off, DB)], ridx)
        pltpu.sync_copy(wod_hbm.at[pl.ds(off, DB)], wodb)
        pltpu.sync_copy(dfx_hbm.at[pl.ds(off, DB)], dbx)
        pltpu.sync_copy(dfy_hbm.at[pl.ds(off, DB)], dby)
        pltpu.sync_copy(dfz_hbm.at[pl.ds(off, DB)], dbz)

        def delta_group(g, carry2):
            o = g * 16
            ir = ridx[pl.ds(o, 16)]
            wod16 = wodb[pl.ds(o, 16)]
            plsc.addupdate_scatter(paccx, [ir], dbx[pl.ds(o, 16)] * wod16)
            plsc.addupdate_scatter(paccy, [ir], dby[pl.ds(o, 16)] * wod16)
            plsc.addupdate_scatter(paccz, [ir], dbz[pl.ds(o, 16)] * wod16)
            return carry2

        lax.fori_loop(0, DB // 16, delta_group, 0)
        return carry

    lax.fori_loop(0, EW2 // DB, superdelta, 0)
    plsc.subcore_barrier()
    pltpu.sync_copy(agg_sh.at[pl.ds(s * NSL, NSL)],
                    agg_hbm.at[c, pl.ds(s * NSL, NSL)])
    pltpu.sync_copy(paccx, px_hbm.at[wid, 0])
    pltpu.sync_copy(paccy, py_hbm.at[wid, 0])
    pltpu.sync_copy(paccz, pz_hbm.at[wid, 0])


def _scatter(msg, wod, dfx, dfy, dfz, row3d, rows, zero_h, zero_p):
    mesh = plsc.VectorSubcoreMesh(core_axis_name="c", subcore_axis_name="s")
    f = pl.kernel(
        _scatter_body,
        mesh=mesh,
        out_type=[
            jax.ShapeDtypeStruct((NC, NP2, H), jnp.float32),
            jax.ShapeDtypeStruct((NW, 1, NP2), jnp.float32),
            jax.ShapeDtypeStruct((NW, 1, NP2), jnp.float32),
            jax.ShapeDtypeStruct((NW, 1, NP2), jnp.float32),
        ],
        scratch_types=[
            pltpu.VMEM((SB, C), jnp.int32),
            pltpu.VMEM((DB,), jnp.int32),
            pltpu.VMEM((C, H), jnp.float32),
            pltpu.VMEM((DB,), jnp.float32),
            pltpu.VMEM((DB,), jnp.float32),
            pltpu.VMEM((DB,), jnp.float32),
            pltpu.VMEM((DB,), jnp.float32),
            pltpu.VMEM((NP2,), jnp.float32),
            pltpu.VMEM((NP2,), jnp.float32),
            pltpu.VMEM((NP2,), jnp.float32),
            pltpu.VMEM_SHARED((NP2, H), jnp.float32),
            pltpu.SemaphoreType.DMA,
        ],
        compiler_params=pltpu.CompilerParams(needs_layout_passes=False),
    )
    return f(msg, wod, dfx, dfy, dfz, row3d, rows, zero_h, zero_p)


def _node_body(h_ref, a0_ref, a1_ref, px_ref, py_ref, pz_ref,
               pix_ref, piy_ref, piz_ref,
               wn1at_ref, wn1bt_ref, bn1_ref, wn2t_ref, bn2_ref,
               hout_ref, sx_ref, sy_ref, sz_ref):
    h = h_ref[...]
    agg = a0_ref[...] + a1_ref[...]
    x1 = (jnp.dot(h, wn1at_ref[...], preferred_element_type=jnp.float32)
          + jnp.dot(agg, wn1bt_ref[...], preferred_element_type=jnp.float32)
          + bn1_ref[...])
    hout_ref[...] = h + (jnp.dot(_silu(x1), wn2t_ref[...],
                                 preferred_element_type=jnp.float32)
                         + bn2_ref[...])
    sx_ref[...] = pix_ref[...] + jnp.sum(px_ref[...].reshape(NW, NP2),
                                         axis=0, keepdims=True)
    sy_ref[...] = piy_ref[...] + jnp.sum(py_ref[...].reshape(NW, NP2),
                                         axis=0, keepdims=True)
    sz_ref[...] = piz_ref[...] + jnp.sum(pz_ref[...].reshape(NW, NP2),
                                         axis=0, keepdims=True)


def _node_mlp(h, a0, a1, pacx, pacy, pacz, pix, piy, piz,
              Wn1aT, Wn1bT, bn1, Wn2T, bn2):
    blk = 2000
    return pl.pallas_call(
        _node_body,
        grid=(N // blk,),
        in_specs=[
            pl.BlockSpec((blk, H), lambda i: (i, 0)),
            pl.BlockSpec((blk, H), lambda i: (i, 0)),
            pl.BlockSpec((blk, H), lambda i: (i, 0)),
            pl.BlockSpec((NW, 1, NP2), lambda i: (0, 0, 0)),
            pl.BlockSpec((NW, 1, NP2), lambda i: (0, 0, 0)),
            pl.BlockSpec((NW, 1, NP2), lambda i: (0, 0, 0)),
            pl.BlockSpec((1, NP2), lambda i: (0, 0)),
            pl.BlockSpec((1, NP2), lambda i: (0, 0)),
            pl.BlockSpec((1, NP2), lambda i: (0, 0)),
            pl.BlockSpec((H, H), lambda i: (0, 0)),
            pl.BlockSpec((H, H), lambda i: (0, 0)),
            pl.BlockSpec((1, H), lambda i: (0, 0)),
            pl.BlockSpec((H, H), lambda i: (0, 0)),
            pl.BlockSpec((1, H), lambda i: (0, 0)),
        ],
        out_specs=[
            pl.BlockSpec((blk, H), lambda i: (i, 0)),
            pl.BlockSpec((1, NP2), lambda i: (0, 0)),
            pl.BlockSpec((1, NP2), lambda i: (0, 0)),
            pl.BlockSpec((1, NP2), lambda i: (0, 0)),
        ],
        out_shape=[
            jax.ShapeDtypeStruct((N, H), jnp.float32),
            jax.ShapeDtypeStruct((1, NP2), jnp.float32),
            jax.ShapeDtypeStruct((1, NP2), jnp.float32),
            jax.ShapeDtypeStruct((1, NP2), jnp.float32),
        ],
    )(h, a0, a1, pacx, pacy, pacz, pix, piy, piz,
      Wn1aT, Wn1bT, bn1, Wn2T, bn2)


def kernel(h, pos, edge_index, We1, be1, We2, be2, Wn1, bn1, Wn2, bn2,
           Wc1, bc1, Wc2):
    f32 = jnp.float32
    i32 = jnp.int32
    WaT = We1[:, :H].T
    WbT = We1[:, H:2 * H].T
    w1d = We1[:, 2 * H].reshape(1, H)
    We2T = We2.T
    Wc1T = Wc1.T
    Wn1aT = Wn1[:, :H].T
    Wn1bT = Wn1[:, H:].T
    Wn2T = Wn2.T
    be1r = be1.reshape(1, H)
    be2r = be2.reshape(1, H)
    bc1r = bc1.reshape(1, H)
    bn1r = bn1.reshape(1, H)
    bn2r = bn2.reshape(1, H)
    Wc2r = Wc2.reshape(1, H)

    px = pos[:, 0]
    py = pos[:, 1]
    pz = pos[:, 2]
    pad = EW2 - EW
    rw = edge_index[0].reshape(NW, EW)
    cw = edge_index[1].reshape(NW, EW)
    zpad = jnp.zeros((NW, pad), dtype=i32)
    rowg = jnp.concatenate([rw, zpad], axis=1).reshape(E2)
    colg = jnp.concatenate([cw, zpad], axis=1).reshape(E2)
    rows = jnp.concatenate([rw, jnp.full((NW, pad), N, dtype=i32)],
                           axis=1).reshape(E2)
    row3d = rows.reshape(NW, K2, C)
    zero_h = jnp.zeros((NP2, H), dtype=f32)
    zero_p = jnp.zeros((NP2,), dtype=f32)
    pixp = jnp.pad(px, (0, NP2 - N)).reshape(1, NP2)
    piyp = jnp.pad(py, (0, NP2 - N)).reshape(1, NP2)
    pizp = jnp.pad(pz, (0, NP2 - N)).reshape(1, NP2)

    Xa, Xb = _precompute(h, WaT, WbT)
    U0, dfx, dfy, dfz = _gather(Xa, Xb, px, py, pz, rowg, colg)
    msg, wod_g = _edge_mlp(U0, dfx, dfy, dfz, w1d, be1r, We2T, be2r,
                           Wc1T, bc1r, Wc2r)
    wod = wod_g.reshape(E2)
    agg2, pacx, pacy, pacz = _scatter(msg, wod, dfx, dfy, dfz, row3d, rows,
                                      zero_h, zero_p)
    h_out, sx, sy, sz = _node_mlp(h, agg2[0, :N], agg2[1, :N],
                                  pacx, pacy, pacz, pixp, piyp, pizp,
                                  Wn1aT, Wn1bT, bn1r, Wn2T, bn2r)
    pos_out = jnp.concatenate([sx[0, :N].reshape(N, 1),
                               sy[0, :N].reshape(N, 1),
                               sz[0, :N].reshape(N, 1)], axis=1)
    return (h_out, pos_out)


# speedup vs baseline: 1.3071x; 1.1327x over previous
import jax
import jax.numpy as jnp
from jax import lax
from jax.experimental import pallas as pl
from jax.experimental.pallas import tpu as pltpu
from jax.experimental.pallas import tpu_sc as plsc

N = 10000
E = 320000
H = 128

NC = 2
NS = 16
NW = NC * NS
EW = E // NW
EW2 = 10240
E2 = EW2 * NW
C = 64
CG = 80
KG = EW2 // CG
K2 = EW2 // C
G2 = EW2 // 16
SB = 16
DB = 1280
NP2 = 10240
NSL = NP2 // NS

BE = 1280
GL = BE // 128
NB = E2 // BE


def _silu(x):
    return x / (1.0 + jnp.exp(-x))


def _precompute_body(h_ref, wa_ref, wb_ref, xa_ref, xb_ref):
    h = h_ref[...]
    xa_ref[...] = jnp.dot(h, wa_ref[...], preferred_element_type=jnp.float32)
    xb_ref[...] = jnp.dot(h, wb_ref[...], preferred_element_type=jnp.float32)


def _precompute(h, WaT, WbT):
    blk = 2000
    return pl.pallas_call(
        _precompute_body,
        grid=(N // blk,),
        in_specs=[
            pl.BlockSpec((blk, H), lambda i: (i, 0)),
            pl.BlockSpec((H, H), lambda i: (0, 0)),
            pl.BlockSpec((H, H), lambda i: (0, 0)),
        ],
        out_specs=[
            pl.BlockSpec((blk, H), lambda i: (i, 0)),
            pl.BlockSpec((blk, H), lambda i: (i, 0)),
        ],
        out_shape=[
            jax.ShapeDtypeStruct((N, H), jnp.float32),
            jax.ShapeDtypeStruct((N, H), jnp.float32),
        ],
    )(h, WaT, WbT)


def _gather_body(xa_hbm, xb_hbm, px_hbm, py_hbm, pz_hbm, rowg_hbm, colg_hbm,
                 u_hbm, dfx_hbm, dfy_hbm, dfz_hbm,
                 ridx, cidx, posx, posy, posz,
                 bufa0, bufb0, bufa1, bufb1, dbx, dby, dbz,
                 gsem0, gsem1, wsem0, wsem1):
    wid = lax.axis_index("s") * NC + lax.axis_index("c")
    base = wid * EW2
    pltpu.sync_copy(rowg_hbm.at[pl.ds(base, EW2)], ridx)
    pltpu.sync_copy(colg_hbm.at[pl.ds(base, EW2)], cidx)
    pltpu.sync_copy(px_hbm, posx)
    pltpu.sync_copy(py_hbm, posy)
    pltpu.sync_copy(pz_hbm, posz)

    def diff_group(g, carry):
        o = g * 16
        ir = ridx[pl.ds(o, 16)]
        ic = cidx[pl.ds(o, 16)]
        dbx[pl.ds(o, 16)] = (plsc.load_gather(posx, [ir])
                             - plsc.load_gather(posx, [ic]))
        dby[pl.ds(o, 16)] = (plsc.load_gather(posy, [ir])
                             - plsc.load_gather(posy, [ic]))
        dbz[pl.ds(o, 16)] = (plsc.load_gather(posz, [ir])
                             - plsc.load_gather(posz, [ic]))
        return carry

    lax.fori_loop(0, G2, diff_group, 0)
    pltpu.sync_copy(dbx, dfx_hbm.at[pl.ds(base, EW2)])
    pltpu.sync_copy(dby, dfy_hbm.at[pl.ds(base, EW2)])
    pltpu.sync_copy(dbz, dfz_hbm.at[pl.ds(base, EW2)])

    bufs = ((bufa0, bufb0, gsem0, wsem0), (bufa1, bufb1, gsem1, wsem1))

    def fire_gather(k, ba, bb, gs):
        ko = k * CG
        pltpu.make_async_copy(xa_hbm.at[ridx.at[pl.ds(ko, CG)]], ba,
                              gs).start()
        pltpu.make_async_copy(xb_hbm.at[cidx.at[pl.ds(ko, CG)]], bb,
                              gs).start()

    def add_rows(ba, bb):
        def body(r, carry):
            for q in range(8):
                o = q * 16
                ba[r, pl.ds(o, 16)] = (ba[r, pl.ds(o, 16)]
                                       + bb[r, pl.ds(o, 16)])
            return carry
        lax.fori_loop(0, CG, body, 0)

    fire_gather(0, bufa0, bufb0, gsem0)

    def step(j, carry):
        for b in (0, 1):
            k = 2 * j + b
            ba, bb, gs, ws = bufs[b]
            nba, nbb, ngs, nws = bufs[1 - b]

            @pl.when(k >= 1)
            def _():
                pltpu.make_async_copy(
                    nba, u_hbm.at[pl.ds(base, CG)], nws).wait()

            @pl.when(k < KG - 1)
            def _():
                fire_gather(k + 1, nba, nbb, ngs)

            pltpu.make_async_copy(
                xa_hbm.at[ridx.at[pl.ds(k * CG, CG)]], ba, gs).wait()
            pltpu.make_async_copy(
                xb_hbm.at[cidx.at[pl.ds(k * CG, CG)]], bb, gs).wait()
            add_rows(ba, bb)
            pltpu.make_async_copy(
                ba, u_hbm.at[pl.ds(base + k * CG, CG)], ws).start()
        return carry

    lax.fori_loop(0, KG // 2, step, 0)
    pltpu.make_async_copy(bufa1, u_hbm.at[pl.ds(base, CG)], wsem1).wait()


def _gather(Xa, Xb, px, py, pz, rowg, colg):
    mesh = plsc.VectorSubcoreMesh(core_axis_name="c", subcore_axis_name="s")
    f = pl.kernel(
        _gather_body,
        mesh=mesh,
        out_type=[
            jax.ShapeDtypeStruct((E2, H), jnp.float32),
            jax.ShapeDtypeStruct((E2,), jnp.float32),
            jax.ShapeDtypeStruct((E2,), jnp.float32),
            jax.ShapeDtypeStruct((E2,), jnp.float32),
        ],
        scratch_types=[
            pltpu.VMEM((EW2,), jnp.int32),
            pltpu.VMEM((EW2,), jnp.int32),
            pltpu.VMEM((N,), jnp.float32),
            pltpu.VMEM((N,), jnp.float32),
            pltpu.VMEM((N,), jnp.float32),
            pltpu.VMEM((CG, H), jnp.float32),
            pltpu.VMEM((CG, H), jnp.float32),
            pltpu.VMEM((CG, H), jnp.float32),
            pltpu.VMEM((CG, H), jnp.float32),
            pltpu.VMEM((EW2,), jnp.float32),
            pltpu.VMEM((EW2,), jnp.float32),
            pltpu.VMEM((EW2,), jnp.float32),
            pltpu.SemaphoreType.DMA,
            pltpu.SemaphoreType.DMA,
            pltpu.SemaphoreType.DMA,
            pltpu.SemaphoreType.DMA,
        ],
        compiler_params=pltpu.CompilerParams(needs_layout_passes=False),
    )
    return f(Xa, Xb, px, py, pz, rowg, colg)


def _edge_body(u0_ref, dfx_ref, dfy_ref, dfz_ref,
               w1d_ref, be1_ref, w2t_ref, be2_ref,
               wc1t_ref, bc1_ref, wc2_ref,
               msg_ref, wod_ref):
    f32 = jnp.float32
    dx = dfx_ref[...].reshape(GL, 128)
    dy = dfy_ref[...].reshape(GL, 128)
    dz = dfz_ref[...].reshape(GL, 128)
    d2 = dx * dx + dy * dy + dz * dz
    dist_g = jnp.maximum(jnp.sqrt(d2), 1e-6)
    dn0 = (((0,), (0,)), ((), ()))
    dist_w1 = jnp.concatenate(
        [lax.dot_general(dist_g[r:r + 1, :], w1d_ref[...], dn0,
                         preferred_element_type=f32) for r in range(GL)],
        axis=0)
    u = u0_ref[...] + dist_w1 + be1_ref[...]
    e1 = _silu(u)
    msg = _silu(jnp.dot(e1, w2t_ref[...], preferred_element_type=jnp.float32)
                + be2_ref[...])
    msg_ref[...] = msg
    t = _silu(jnp.dot(msg, wc1t_ref[...], preferred_element_type=jnp.float32)
              + bc1_ref[...])
    dn1 = (((1,), (1,)), ((), ()))
    w_g = jnp.concatenate(
        [lax.dot_general(wc2_ref[...], t[r * 128:(r + 1) * 128, :], dn1,
                         preferred_element_type=f32) for r in range(GL)],
        axis=0)
    w_g = jnp.clip(w_g, -1.0, 1.0)
    wod_ref[...] = (w_g / dist_g).reshape(1, GL, 128)


def _edge_mlp(U0, dfx, dfy, dfz, w1d, be1, We2T, be2, Wc1T, bc1, Wc2):
    return pl.pallas_call(
        _edge_body,
        grid=(NB,),
        in_specs=[
            pl.BlockSpec((BE, H), lambda i: (i, 0)),
            pl.BlockSpec((1, GL, 128), lambda i: (i, 0, 0)),
            pl.BlockSpec((1, GL, 128), lambda i: (i, 0, 0)),
            pl.BlockSpec((1, GL, 128), lambda i: (i, 0, 0)),
            pl.BlockSpec((1, H), lambda i: (0, 0)),
            pl.BlockSpec((1, H), lambda i: (0, 0)),
            pl.BlockSpec((H, H), lambda i: (0, 0)),
            pl.BlockSpec((1, H), lambda i: (0, 0)),
            pl.BlockSpec((H, H), lambda i: (0, 0)),
            pl.BlockSpec((1, H), lambda i: (0, 0)),
            pl.BlockSpec((1, H), lambda i: (0, 0)),
        ],
        out_specs=[
            pl.BlockSpec((BE, H), lambda i: (i, 0)),
            pl.BlockSpec((1, GL, 128), lambda i: (i, 0, 0)),
        ],
        out_shape=[
            jax.ShapeDtypeStruct((E2, H), jnp.float32),
            jax.ShapeDtypeStruct((NB, GL, 128), jnp.float32),
        ],
    )(U0, dfx.reshape(NB, GL, 128), dfy.reshape(NB, GL, 128),
      dfz.reshape(NB, GL, 128), w1d, be1, We2T, be2, Wc1T, bc1, Wc2)


def _scatter_body(msg_hbm, wod_hbm, dfx_hbm, dfy_hbm, dfz_hbm,
                  row3d_hbm, rows_hbm, zh_hbm, zp_hbm,
                  agg_hbm, px_hbm, py_hbm, pz_hbm,
                  ridx2, ridx, mbuf, dbx, dby, dbz, wodb,
                  paccx, paccy, paccz, agg_sh, lsem):
    c = lax.axis_index("c")
    s = lax.axis_index("s")
    wid = s * NC + c
    base = wid * EW2
    pltpu.sync_copy(zh_hbm.at[pl.ds(s * NSL, NSL)],
                    agg_sh.at[pl.ds(s * NSL, NSL)])
    pltpu.sync_copy(zp_hbm, paccx)
    pltpu.sync_copy(zp_hbm, paccy)
    pltpu.sync_copy(zp_hbm, paccz)
    plsc.subcore_barrier()

    def superchunk(sb, carry):
        pltpu.sync_copy(row3d_hbm.at[wid, pl.ds(sb * SB, SB)], ridx2)

        def chunk(k, carry2):
            off = base + (sb * SB + k) * C
            lm = pltpu.make_async_copy(msg_hbm.at[pl.ds(off, C)], mbuf,
                                       lsem)
            lm.start()
            lm.wait()
            pltpu.sync_copy(mbuf, agg_sh.at[ridx2.at[k]], add=True)
            return carry2

        lax.fori_loop(0, SB, chunk, 0)
        return carry

    lax.fori_loop(0, K2 // SB, superchunk, 0)

    def superdelta(j, carry):
        off = base + j * DB
        pltpu.sync_copy(rows_hbm.at[pl.ds(off, DB)], ridx)
        pltpu.sync_copy(wod_hbm.at[pl.ds(off, DB)], wodb)
        pltpu.sync_copy(dfx_hbm.at[pl.ds(off, DB)], dbx)
        pltpu.sync_copy(dfy_hbm.at[pl.ds(off, DB)], dby)
        pltpu.sync_copy(dfz_hbm.at[pl.ds(off, DB)], dbz)

        def delta_group(g, carry2):
            o = g * 16
            ir = ridx[pl.ds(o, 16)]
            wod16 = wodb[pl.ds(o, 16)]
            plsc.addupdate_scatter(paccx, [ir], dbx[pl.ds(o, 16)] * wod16)
            plsc.addupdate_scatter(paccy, [ir], dby[pl.ds(o, 16)] * wod16)
            plsc.addupdate_scatter(paccz, [ir], dbz[pl.ds(o, 16)] * wod16)
            return carry2

        lax.fori_loop(0, DB // 16, delta_group, 0)
        return carry

    lax.fori_loop(0, EW2 // DB, superdelta, 0)
    plsc.subcore_barrier()
    pltpu.sync_copy(agg_sh.at[pl.ds(s * NSL, NSL)],
                    agg_hbm.at[c, pl.ds(s * NSL, NSL)])
    pltpu.sync_copy(paccx, px_hbm.at[wid, 0])
    pltpu.sync_copy(paccy, py_hbm.at[wid, 0])
    pltpu.sync_copy(paccz, pz_hbm.at[wid, 0])


def _scatter(msg, wod, dfx, dfy, dfz, row3d, rows, zero_h, zero_p):
    mesh = plsc.VectorSubcoreMesh(core_axis_name="c", subcore_axis_name="s")
    f = pl.kernel(
        _scatter_body,
        mesh=mesh,
        out_type=[
            jax.ShapeDtypeStruct((NC, NP2, H), jnp.float32),
            jax.ShapeDtypeStruct((NW, 1, NP2), jnp.float32),
            jax.ShapeDtypeStruct((NW, 1, NP2), jnp.float32),
            jax.ShapeDtypeStruct((NW, 1, NP2), jnp.float32),
        ],
        scratch_types=[
            pltpu.VMEM((SB, C), jnp.int32),
            pltpu.VMEM((DB,), jnp.int32),
            pltpu.VMEM((C, H), jnp.float32),
            pltpu.VMEM((DB,), jnp.float32),
            pltpu.VMEM((DB,), jnp.float32),
            pltpu.VMEM((DB,), jnp.float32),
            pltpu.VMEM((DB,), jnp.float32),
            pltpu.VMEM((NP2,), jnp.float32),
            pltpu.VMEM((NP2,), jnp.float32),
            pltpu.VMEM((NP2,), jnp.float32),
            pltpu.VMEM_SHARED((NP2, H), jnp.float32),
            pltpu.SemaphoreType.DMA,
        ],
        compiler_params=pltpu.CompilerParams(needs_layout_passes=False),
    )
    return f(msg, wod, dfx, dfy, dfz, row3d, rows, zero_h, zero_p)


def _node_body(h_ref, a0_ref, a1_ref, px_ref, py_ref, pz_ref,
               pix_ref, piy_ref, piz_ref,
               wn1at_ref, wn1bt_ref, bn1_ref, wn2t_ref, bn2_ref,
               hout_ref, sx_ref, sy_ref, sz_ref):
    h = h_ref[...]
    agg = a0_ref[...] + a1_ref[...]
    x1 = (jnp.dot(h, wn1at_ref[...], preferred_element_type=jnp.float32)
          + jnp.dot(agg, wn1bt_ref[...], preferred_element_type=jnp.float32)
          + bn1_ref[...])
    hout_ref[...] = h + (jnp.dot(_silu(x1), wn2t_ref[...],
                                 preferred_element_type=jnp.float32)
                         + bn2_ref[...])
    sx_ref[...] = pix_ref[...] + jnp.sum(px_ref[...].reshape(NW, NP2),
                                         axis=0, keepdims=True)
    sy_ref[...] = piy_ref[...] + jnp.sum(py_ref[...].reshape(NW, NP2),
                                         axis=0, keepdims=True)
    sz_ref[...] = piz_ref[...] + jnp.sum(pz_ref[...].reshape(NW, NP2),
                                         axis=0, keepdims=True)


def _node_mlp(h, a0, a1, pacx, pacy, pacz, pix, piy, piz,
              Wn1aT, Wn1bT, bn1, Wn2T, bn2):
    blk = 2000
    return pl.pallas_call(
        _node_body,
        grid=(N // blk,),
        in_specs=[
            pl.BlockSpec((blk, H), lambda i: (i, 0)),
            pl.BlockSpec((blk, H), lambda i: (i, 0)),
            pl.BlockSpec((blk, H), lambda i: (i, 0)),
            pl.BlockSpec((NW, 1, NP2), lambda i: (0, 0, 0)),
            pl.BlockSpec((NW, 1, NP2), lambda i: (0, 0, 0)),
            pl.BlockSpec((NW, 1, NP2), lambda i: (0, 0, 0)),
            pl.BlockSpec((1, NP2), lambda i: (0, 0)),
            pl.BlockSpec((1, NP2), lambda i: (0, 0)),
            pl.BlockSpec((1, NP2), lambda i: (0, 0)),
            pl.BlockSpec((H, H), lambda i: (0, 0)),
            pl.BlockSpec((H, H), lambda i: (0, 0)),
            pl.BlockSpec((1, H), lambda i: (0, 0)),
            pl.BlockSpec((H, H), lambda i: (0, 0)),
            pl.BlockSpec((1, H), lambda i: (0, 0)),
        ],
        out_specs=[
            pl.BlockSpec((blk, H), lambda i: (i, 0)),
            pl.BlockSpec((1, NP2), lambda i: (0, 0)),
            pl.BlockSpec((1, NP2), lambda i: (0, 0)),
            pl.BlockSpec((1, NP2), lambda i: (0, 0)),
        ],
        out_shape=[
            jax.ShapeDtypeStruct((N, H), jnp.float32),
            jax.ShapeDtypeStruct((1, NP2), jnp.float32),
            jax.ShapeDtypeStruct((1, NP2), jnp.float32),
            jax.ShapeDtypeStruct((1, NP2), jnp.float32),
        ],
    )(h, a0, a1, pacx, pacy, pacz, pix, piy, piz,
      Wn1aT, Wn1bT, bn1, Wn2T, bn2)


def kernel(h, pos, edge_index, We1, be1, We2, be2, Wn1, bn1, Wn2, bn2,
           Wc1, bc1, Wc2):
    f32 = jnp.float32
    i32 = jnp.int32
    WaT = We1[:, :H].T
    WbT = We1[:, H:2 * H].T
    w1d = We1[:, 2 * H].reshape(1, H)
    We2T = We2.T
    Wc1T = Wc1.T
    Wn1aT = Wn1[:, :H].T
    Wn1bT = Wn1[:, H:].T
    Wn2T = Wn2.T
    be1r = be1.reshape(1, H)
    be2r = be2.reshape(1, H)
    bc1r = bc1.reshape(1, H)
    bn1r = bn1.reshape(1, H)
    bn2r = bn2.reshape(1, H)
    Wc2r = Wc2.reshape(1, H)

    px = pos[:, 0]
    py = pos[:, 1]
    pz = pos[:, 2]
    pad = EW2 - EW
    rw = edge_index[0].reshape(NW, EW)
    cw = edge_index[1].reshape(NW, EW)
    zpad = jnp.zeros((NW, pad), dtype=i32)
    rowg = jnp.concatenate([rw, zpad], axis=1).reshape(E2)
    colg = jnp.concatenate([cw, zpad], axis=1).reshape(E2)
    rows = jnp.concatenate([rw, jnp.full((NW, pad), N, dtype=i32)],
                           axis=1).reshape(E2)
    row3d = rows.reshape(NW, K2, C)
    zero_h = jnp.zeros((NP2, H), dtype=f32)
    zero_p = jnp.zeros((NP2,), dtype=f32)
    pixp = jnp.pad(px, (0, NP2 - N)).reshape(1, NP2)
    piyp = jnp.pad(py, (0, NP2 - N)).reshape(1, NP2)
    pizp = jnp.pad(pz, (0, NP2 - N)).reshape(1, NP2)

    Xa, Xb = _precompute(h, WaT, WbT)
    U0, dfx, dfy, dfz = _gather(Xa, Xb, px, py, pz, rowg, colg)
    msg, wod_g = _edge_mlp(U0, dfx, dfy, dfz, w1d, be1r, We2T, be2r,
                           Wc1T, bc1r, Wc2r)
    wod = wod_g.reshape(E2)
    agg2, pacx, pacy, pacz = _scatter(msg, wod, dfx, dfy, dfz, row3d, rows,
                                      zero_h, zero_p)
    h_out, sx, sy, sz = _node_mlp(h, agg2[0, :N], agg2[1, :N],
                                  pacx, pacy, pacz, pixp, piyp, pizp,
                                  Wn1aT, Wn1bT, bn1r, Wn2T, bn2r)
    pos_out = jnp.concatenate([sx[0, :N].reshape(N, 1),
                               sy[0, :N].reshape(N, 1),
                               sz[0, :N].reshape(N, 1)], axis=1)
    return (h_out, pos_out)


# speedup vs baseline: 1.5102x; 1.1554x over previous
import jax
import jax.numpy as jnp
from jax import lax
from jax.experimental import pallas as pl
from jax.experimental.pallas import tpu as pltpu
from jax.experimental.pallas import tpu_sc as plsc

N = 10000
E = 320000
H = 128

NC = 2
NS = 16
NW = NC * NS
EW = E // NW
EW2 = 10240
E2 = EW2 * NW
C = 64
CG = 80
KG = EW2 // CG
K2 = EW2 // C
G2 = EW2 // 16
CS = 128
KS = EW2 // CS
DB = 2560
NP2 = 10240
NSL = NP2 // NS

BE = 1280
GL = BE // 128
NB = E2 // BE


def _silu(x):
    return x / (1.0 + jnp.exp(-x))


def _precompute_body(h_ref, wa_ref, wb_ref, xa_ref, xb_ref):
    h = h_ref[...]
    xa_ref[...] = jnp.dot(h, wa_ref[...], preferred_element_type=jnp.float32)
    xb_ref[...] = jnp.dot(h, wb_ref[...], preferred_element_type=jnp.float32)


def _precompute(h, WaT, WbT):
    blk = 2000
    return pl.pallas_call(
        _precompute_body,
        grid=(N // blk,),
        in_specs=[
            pl.BlockSpec((blk, H), lambda i: (i, 0)),
            pl.BlockSpec((H, H), lambda i: (0, 0)),
            pl.BlockSpec((H, H), lambda i: (0, 0)),
        ],
        out_specs=[
            pl.BlockSpec((blk, H), lambda i: (i, 0)),
            pl.BlockSpec((blk, H), lambda i: (i, 0)),
        ],
        out_shape=[
            jax.ShapeDtypeStruct((N, H), jnp.float32),
            jax.ShapeDtypeStruct((N, H), jnp.float32),
        ],
    )(h, WaT, WbT)


def _gather_body(xa_hbm, xb_hbm, px_hbm, py_hbm, pz_hbm, rowg_hbm, colg_hbm,
                 u_hbm, dfx_hbm, dfy_hbm, dfz_hbm,
                 ridx, cidx, posx, posy, posz,
                 bufa0, bufb0, bufa1, bufb1, dbx, dby, dbz,
                 gsem0, gsem1, wsem0, wsem1):
    wid = lax.axis_index("s") * NC + lax.axis_index("c")
    base = wid * EW2
    pltpu.sync_copy(rowg_hbm.at[pl.ds(base, EW2)], ridx)
    pltpu.sync_copy(colg_hbm.at[pl.ds(base, EW2)], cidx)
    pltpu.sync_copy(px_hbm, posx)
    pltpu.sync_copy(py_hbm, posy)
    pltpu.sync_copy(pz_hbm, posz)

    def diff_group(g, carry):
        o = g * 16
        ir = ridx[pl.ds(o, 16)]
        ic = cidx[pl.ds(o, 16)]
        dbx[pl.ds(o, 16)] = (plsc.load_gather(posx, [ir])
                             - plsc.load_gather(posx, [ic]))
        dby[pl.ds(o, 16)] = (plsc.load_gather(posy, [ir])
                             - plsc.load_gather(posy, [ic]))
        dbz[pl.ds(o, 16)] = (plsc.load_gather(posz, [ir])
                             - plsc.load_gather(posz, [ic]))
        return carry

    lax.fori_loop(0, G2, diff_group, 0)
    pltpu.sync_copy(dbx, dfx_hbm.at[pl.ds(base, EW2)])
    pltpu.sync_copy(dby, dfy_hbm.at[pl.ds(base, EW2)])
    pltpu.sync_copy(dbz, dfz_hbm.at[pl.ds(base, EW2)])

    bufs = ((bufa0, bufb0, gsem0, wsem0), (bufa1, bufb1, gsem1, wsem1))

    def fire_gather(k, ba, bb, gs):
        ko = k * CG
        pltpu.make_async_copy(xa_hbm.at[ridx.at[pl.ds(ko, CG)]], ba,
                              gs).start()
        pltpu.make_async_copy(xb_hbm.at[cidx.at[pl.ds(ko, CG)]], bb,
                              gs).start()

    def add_rows(ba, bb):
        def body(r, carry):
            for q in range(8):
                o = q * 16
                ba[r, pl.ds(o, 16)] = (ba[r, pl.ds(o, 16)]
                                       + bb[r, pl.ds(o, 16)])
            return carry
        lax.fori_loop(0, CG, body, 0)

    fire_gather(0, bufa0, bufb0, gsem0)

    def step(j, carry):
        for b in (0, 1):
            k = 2 * j + b
            ba, bb, gs, ws = bufs[b]
            nba, nbb, ngs, nws = bufs[1 - b]

            @pl.when(k >= 1)
            def _():
                pltpu.make_async_copy(
                    nba, u_hbm.at[pl.ds(base, CG)], nws).wait()

            @pl.when(k < KG - 1)
            def _():
                fire_gather(k + 1, nba, nbb, ngs)

            pltpu.make_async_copy(
                xa_hbm.at[ridx.at[pl.ds(k * CG, CG)]], ba, gs).wait()
            pltpu.make_async_copy(
                xb_hbm.at[cidx.at[pl.ds(k * CG, CG)]], bb, gs).wait()
            add_rows(ba, bb)
            pltpu.make_async_copy(
                ba, u_hbm.at[pl.ds(base + k * CG, CG)], ws).start()
        return carry

    lax.fori_loop(0, KG // 2, step, 0)
    pltpu.make_async_copy(bufa1, u_hbm.at[pl.ds(base, CG)], wsem1).wait()


def _gather(Xa, Xb, px, py, pz, rowg, colg):
    mesh = plsc.VectorSubcoreMesh(core_axis_name="c", subcore_axis_name="s")
    f = pl.kernel(
        _gather_body,
        mesh=mesh,
        out_type=[
            jax.ShapeDtypeStruct((E2, H), jnp.float32),
            jax.ShapeDtypeStruct((E2,), jnp.float32),
            jax.ShapeDtypeStruct((E2,), jnp.float32),
            jax.ShapeDtypeStruct((E2,), jnp.float32),
        ],
        scratch_types=[
            pltpu.VMEM((EW2,), jnp.int32),
            pltpu.VMEM((EW2,), jnp.int32),
            pltpu.VMEM((N,), jnp.float32),
            pltpu.VMEM((N,), jnp.float32),
            pltpu.VMEM((N,), jnp.float32),
            pltpu.VMEM((CG, H), jnp.float32),
            pltpu.VMEM((CG, H), jnp.float32),
            pltpu.VMEM((CG, H), jnp.float32),
            pltpu.VMEM((CG, H), jnp.float32),
            pltpu.VMEM((EW2,), jnp.float32),
            pltpu.VMEM((EW2,), jnp.float32),
            pltpu.VMEM((EW2,), jnp.float32),
            pltpu.SemaphoreType.DMA,
            pltpu.SemaphoreType.DMA,
            pltpu.SemaphoreType.DMA,
            pltpu.SemaphoreType.DMA,
        ],
        compiler_params=pltpu.CompilerParams(needs_layout_passes=False),
    )
    return f(Xa, Xb, px, py, pz, rowg, colg)


def _edge_body(u0_ref, dfx_ref, dfy_ref, dfz_ref,
               w1d_ref, be1_ref, w2t_ref, be2_ref,
               wc1t_ref, bc1_ref, wc2_ref,
               msg_ref, wod_ref):
    f32 = jnp.float32
    dx = dfx_ref[...].reshape(GL, 128)
    dy = dfy_ref[...].reshape(GL, 128)
    dz = dfz_ref[...].reshape(GL, 128)
    d2 = dx * dx + dy * dy + dz * dz
    dist_g = jnp.maximum(jnp.sqrt(d2), 1e-6)
    dn0 = (((0,), (0,)), ((), ()))
    dist_w1 = jnp.concatenate(
        [lax.dot_general(dist_g[r:r + 1, :], w1d_ref[...], dn0,
                         preferred_element_type=f32) for r in range(GL)],
        axis=0)
    u = u0_ref[...] + dist_w1 + be1_ref[...]
    e1 = _silu(u)
    msg = _silu(jnp.dot(e1, w2t_ref[...], preferred_element_type=jnp.float32)
                + be2_ref[...])
    msg_ref[...] = msg
    t = _silu(jnp.dot(msg, wc1t_ref[...], preferred_element_type=jnp.float32)
              + bc1_ref[...])
    dn1 = (((1,), (1,)), ((), ()))
    w_g = jnp.concatenate(
        [lax.dot_general(wc2_ref[...], t[r * 128:(r + 1) * 128, :], dn1,
                         preferred_element_type=f32) for r in range(GL)],
        axis=0)
    w_g = jnp.clip(w_g, -1.0, 1.0)
    wod_ref[...] = (w_g / dist_g).reshape(1, GL, 128)


def _edge_mlp(U0, dfx, dfy, dfz, w1d, be1, We2T, be2, Wc1T, bc1, Wc2):
    return pl.pallas_call(
        _edge_body,
        grid=(NB,),
        in_specs=[
            pl.BlockSpec((BE, H), lambda i: (i, 0)),
            pl.BlockSpec((1, GL, 128), lambda i: (i, 0, 0)),
            pl.BlockSpec((1, GL, 128), lambda i: (i, 0, 0)),
            pl.BlockSpec((1, GL, 128), lambda i: (i, 0, 0)),
            pl.BlockSpec((1, H), lambda i: (0, 0)),
            pl.BlockSpec((1, H), lambda i: (0, 0)),
            pl.BlockSpec((H, H), lambda i: (0, 0)),
            pl.BlockSpec((1, H), lambda i: (0, 0)),
            pl.BlockSpec((H, H), lambda i: (0, 0)),
            pl.BlockSpec((1, H), lambda i: (0, 0)),
            pl.BlockSpec((1, H), lambda i: (0, 0)),
        ],
        out_specs=[
            pl.BlockSpec((BE, H), lambda i: (i, 0)),
            pl.BlockSpec((1, GL, 128), lambda i: (i, 0, 0)),
        ],
        out_shape=[
            jax.ShapeDtypeStruct((E2, H), jnp.float32),
            jax.ShapeDtypeStruct((NB, GL, 128), jnp.float32),
        ],
    )(U0, dfx.reshape(NB, GL, 128), dfy.reshape(NB, GL, 128),
      dfz.reshape(NB, GL, 128), w1d, be1, We2T, be2, Wc1T, bc1, Wc2)


def _scatter_agg_body(msg_hbm, row3d_hbm, zh_hbm, agg_hbm,
                      ridx2, mbuf0, mbuf1, agg_sh, lsem0, lsem1):
    c = lax.axis_index("c")
    s = lax.axis_index("s")
    wid = s * NC + c
    base = wid * EW2
    pltpu.sync_copy(zh_hbm.at[pl.ds(s * NSL, NSL)],
                    agg_sh.at[pl.ds(s * NSL, NSL)])
    pltpu.sync_copy(row3d_hbm.at[wid], ridx2)
    plsc.subcore_barrier()

    bufs = ((mbuf0, lsem0), (mbuf1, lsem1))
    pltpu.make_async_copy(msg_hbm.at[pl.ds(base, CS)], mbuf0, lsem0).start()

    def step(j, carry):
        for b in (0, 1):
            k = 2 * j + b
            mb, ls = bufs[b]
            nmb, nls = bufs[1 - b]

            @pl.when(k < KS - 1)
            def _():
                off = base + (k + 1) * CS
                pltpu.make_async_copy(msg_hbm.at[pl.ds(off, CS)], nmb,
                                      nls).start()

            pltpu.make_async_copy(msg_hbm.at[pl.ds(base, CS)], mb, ls
                                  ).wait()
            pltpu.sync_copy(mb, agg_sh.at[ridx2.at[k]], add=True)
        return carry

    lax.fori_loop(0, KS // 2, step, 0)
    plsc.subcore_barrier()
    pltpu.sync_copy(agg_sh.at[pl.ds(s * NSL, NSL)],
                    agg_hbm.at[c, pl.ds(s * NSL, NSL)])


def _scatter_agg(msg, row3s, zero_h):
    mesh = plsc.VectorSubcoreMesh(core_axis_name="c", subcore_axis_name="s")
    f = pl.kernel(
        _scatter_agg_body,
        mesh=mesh,
        out_type=jax.ShapeDtypeStruct((NC, NP2, H), jnp.float32),
        scratch_types=[
            pltpu.VMEM((KS, CS), jnp.int32),
            pltpu.VMEM((CS, H), jnp.float32),
            pltpu.VMEM((CS, H), jnp.float32),
            pltpu.VMEM_SHARED((NP2, H), jnp.float32),
            pltpu.SemaphoreType.DMA,
            pltpu.SemaphoreType.DMA,
        ],
        compiler_params=pltpu.CompilerParams(needs_layout_passes=False),
    )
    return f(msg, row3s, zero_h)


def _scatter_pos_body(wod_hbm, dfx_hbm, dfy_hbm, dfz_hbm, rows_hbm, zp_hbm,
                      px_hbm, py_hbm, pz_hbm,
                      ridx, dbx, dby, dbz, wodb, paccx, paccy, paccz):
    c = lax.axis_index("c")
    s = lax.axis_index("s")
    wid = s * NC + c
    base = wid * EW2
    pltpu.sync_copy(zp_hbm, paccx)
    pltpu.sync_copy(zp_hbm, paccy)
    pltpu.sync_copy(zp_hbm, paccz)

    def superdelta(j, carry):
        off = base + j * DB
        pltpu.sync_copy(rows_hbm.at[pl.ds(off, DB)], ridx)
        pltpu.sync_copy(wod_hbm.at[pl.ds(off, DB)], wodb)
        pltpu.sync_copy(dfx_hbm.at[pl.ds(off, DB)], dbx)
        pltpu.sync_copy(dfy_hbm.at[pl.ds(off, DB)], dby)
        pltpu.sync_copy(dfz_hbm.at[pl.ds(off, DB)], dbz)

        def delta_group(g, carry2):
            o = g * 16
            ir = ridx[pl.ds(o, 16)]
            wod16 = wodb[pl.ds(o, 16)]
            plsc.addupdate_scatter(paccx, [ir], dbx[pl.ds(o, 16)] * wod16)
            plsc.addupdate_scatter(paccy, [ir], dby[pl.ds(o, 16)] * wod16)
            plsc.addupdate_scatter(paccz, [ir], dbz[pl.ds(o, 16)] * wod16)
            return carry2

        lax.fori_loop(0, DB // 16, delta_group, 0)
        return carry

    lax.fori_loop(0, EW2 // DB, superdelta, 0)
    pltpu.sync_copy(paccx, px_hbm.at[wid, 0])
    pltpu.sync_copy(paccy, py_hbm.at[wid, 0])
    pltpu.sync_copy(paccz, pz_hbm.at[wid, 0])


def _scatter_pos(wod, dfx, dfy, dfz, rows, zero_p):
    mesh = plsc.VectorSubcoreMesh(core_axis_name="c", subcore_axis_name="s")
    f = pl.kernel(
        _scatter_pos_body,
        mesh=mesh,
        out_type=[
            jax.ShapeDtypeStruct((NW, 1, NP2), jnp.float32),
            jax.ShapeDtypeStruct((NW, 1, NP2), jnp.float32),
            jax.ShapeDtypeStruct((NW, 1, NP2), jnp.float32),
        ],
        scratch_types=[
            pltpu.VMEM((DB,), jnp.int32),
            pltpu.VMEM((DB,), jnp.float32),
            pltpu.VMEM((DB,), jnp.float32),
            pltpu.VMEM((DB,), jnp.float32),
            pltpu.VMEM((DB,), jnp.float32),
            pltpu.VMEM((NP2,), jnp.float32),
            pltpu.VMEM((NP2,), jnp.float32),
            pltpu.VMEM((NP2,), jnp.float32),
        ],
        compiler_params=pltpu.CompilerParams(needs_layout_passes=False),
    )
    return f(wod, dfx, dfy, dfz, rows, zero_p)


def _node_body(h_ref, a0_ref, a1_ref, px_ref, py_ref, pz_ref,
               pix_ref, piy_ref, piz_ref,
               wn1at_ref, wn1bt_ref, bn1_ref, wn2t_ref, bn2_ref,
               hout_ref, sx_ref, sy_ref, sz_ref):
    h = h_ref[...]
    agg = a0_ref[...] + a1_ref[...]
    x1 = (jnp.dot(h, wn1at_ref[...], preferred_element_type=jnp.float32)
          + jnp.dot(agg, wn1bt_ref[...], preferred_element_type=jnp.float32)
          + bn1_ref[...])
    hout_ref[...] = h + (jnp.dot(_silu(x1), wn2t_ref[...],
                                 preferred_element_type=jnp.float32)
                         + bn2_ref[...])
    sx_ref[...] = pix_ref[...] + jnp.sum(px_ref[...].reshape(NW, NP2),
                                         axis=0, keepdims=True)
    sy_ref[...] = piy_ref[...] + jnp.sum(py_ref[...].reshape(NW, NP2),
                                         axis=0, keepdims=True)
    sz_ref[...] = piz_ref[...] + jnp.sum(pz_ref[...].reshape(NW, NP2),
                                         axis=0, keepdims=True)


def _node_mlp(h, a0, a1, pacx, pacy, pacz, pix, piy, piz,
              Wn1aT, Wn1bT, bn1, Wn2T, bn2):
    blk = 2000
    return pl.pallas_call(
        _node_body,
        grid=(N // blk,),
        in_specs=[
            pl.BlockSpec((blk, H), lambda i: (i, 0)),
            pl.BlockSpec((blk, H), lambda i: (i, 0)),
            pl.BlockSpec((blk, H), lambda i: (i, 0)),
            pl.BlockSpec((NW, 1, NP2), lambda i: (0, 0, 0)),
            pl.BlockSpec((NW, 1, NP2), lambda i: (0, 0, 0)),
            pl.BlockSpec((NW, 1, NP2), lambda i: (0, 0, 0)),
            pl.BlockSpec((1, NP2), lambda i: (0, 0)),
            pl.BlockSpec((1, NP2), lambda i: (0, 0)),
            pl.BlockSpec((1, NP2), lambda i: (0, 0)),
            pl.BlockSpec((H, H), lambda i: (0, 0)),
            pl.BlockSpec((H, H), lambda i: (0, 0)),
            pl.BlockSpec((1, H), lambda i: (0, 0)),
            pl.BlockSpec((H, H), lambda i: (0, 0)),
            pl.BlockSpec((1, H), lambda i: (0, 0)),
        ],
        out_specs=[
            pl.BlockSpec((blk, H), lambda i: (i, 0)),
            pl.BlockSpec((1, NP2), lambda i: (0, 0)),
            pl.BlockSpec((1, NP2), lambda i: (0, 0)),
            pl.BlockSpec((1, NP2), lambda i: (0, 0)),
        ],
        out_shape=[
            jax.ShapeDtypeStruct((N, H), jnp.float32),
            jax.ShapeDtypeStruct((1, NP2), jnp.float32),
            jax.ShapeDtypeStruct((1, NP2), jnp.float32),
            jax.ShapeDtypeStruct((1, NP2), jnp.float32),
        ],
    )(h, a0, a1, pacx, pacy, pacz, pix, piy, piz,
      Wn1aT, Wn1bT, bn1, Wn2T, bn2)


def kernel(h, pos, edge_index, We1, be1, We2, be2, Wn1, bn1, Wn2, bn2,
           Wc1, bc1, Wc2):
    f32 = jnp.float32
    i32 = jnp.int32
    WaT = We1[:, :H].T
    WbT = We1[:, H:2 * H].T
    w1d = We1[:, 2 * H].reshape(1, H)
    We2T = We2.T
    Wc1T = Wc1.T
    Wn1aT = Wn1[:, :H].T
    Wn1bT = Wn1[:, H:].T
    Wn2T = Wn2.T
    be1r = be1.reshape(1, H)
    be2r = be2.reshape(1, H)
    bc1r = bc1.reshape(1, H)
    bn1r = bn1.reshape(1, H)
    bn2r = bn2.reshape(1, H)
    Wc2r = Wc2.reshape(1, H)

    px = pos[:, 0]
    py = pos[:, 1]
    pz = pos[:, 2]
    pad = EW2 - EW
    rw = edge_index[0].reshape(NW, EW)
    cw = edge_index[1].reshape(NW, EW)
    zpad = jnp.zeros((NW, pad), dtype=i32)
    rowg = jnp.concatenate([rw, zpad], axis=1).reshape(E2)
    colg = jnp.concatenate([cw, zpad], axis=1).reshape(E2)
    rows = jnp.concatenate([rw, jnp.full((NW, pad), N, dtype=i32)],
                           axis=1).reshape(E2)
    row3s = rows.reshape(NW, KS, CS)
    zero_h = jnp.zeros((NP2, H), dtype=f32)
    zero_p = jnp.zeros((NP2,), dtype=f32)
    pixp = jnp.pad(px, (0, NP2 - N)).reshape(1, NP2)
    piyp = jnp.pad(py, (0, NP2 - N)).reshape(1, NP2)
    pizp = jnp.pad(pz, (0, NP2 - N)).reshape(1, NP2)

    Xa, Xb = _precompute(h, WaT, WbT)
    U0, dfx, dfy, dfz = _gather(Xa, Xb, px, py, pz, rowg, colg)
    msg, wod_g = _edge_mlp(U0, dfx, dfy, dfz, w1d, be1r, We2T, be2r,
                           Wc1T, bc1r, Wc2r)
    wod = wod_g.reshape(E2)
    agg2 = _scatter_agg(msg, row3s, zero_h)
    pacx, pacy, pacz = _scatter_pos(wod, dfx, dfy, dfz, rows, zero_p)
    h_out, sx, sy, sz = _node_mlp(h, agg2[0, :N], agg2[1, :N],
                                  pacx, pacy, pacz, pixp, piyp, pizp,
                                  Wn1aT, Wn1bT, bn1r, Wn2T, bn2r)
    pos_out = jnp.concatenate([sx[0, :N].reshape(N, 1),
                               sy[0, :N].reshape(N, 1),
                               sz[0, :N].reshape(N, 1)], axis=1)
    return (h_out, pos_out)
